# Initial kernel scaffold; baseline (speedup 1.0000x reference)
import functools

import jax
import jax.numpy as jnp
from jax import lax
from jax.experimental import pallas as pl
from jax.experimental.pallas import tpu as pltpu
from jax.experimental.pallas import tpu_sc as plsc

_N = 10000
_E = 320000
_IN = 128
_OUT = 32
_H = 4
_TW = _H * _OUT + 16
_B = 128
_CHUNKS = _E // _B
_NC = 2
_NS = 16
_NW = _NC * _NS
_RPT = _N // _NS
_NEG = -3.0e38


def _prep_body(x_ref, w_ref, al_ref, ar_ref, tab_ref, er_ref, elmax_ref):
    i = pl.program_id(0)
    x = x_ref[...]
    feats = []
    els = []
    ers = []
    for h in range(_H):
        f = jnp.dot(x, w_ref[h], preferred_element_type=jnp.float32)
        feats.append(f)
        els.append(jnp.sum(f * al_ref[h][None, :], axis=1, keepdims=True))
        ers.append(jnp.sum(f * ar_ref[h][None, :], axis=1, keepdims=True))
    rows = x.shape[0]
    pad12 = jnp.zeros((rows, 12), jnp.float32)
    tab_ref[...] = jnp.concatenate(feats + els + [pad12], axis=1)
    er_ref[...] = jnp.concatenate(ers + [pad12], axis=1)

    el4 = jnp.concatenate(els, axis=1)
    padded = jnp.concatenate(
        [el4, jnp.full((rows, 124), _NEG, jnp.float32)], axis=1)
    blockmax = jnp.max(padded, axis=0, keepdims=True)

    @pl.when(i == 0)
    def _():
        elmax_ref[...] = jnp.full((1, 128), _NEG, jnp.float32)

    elmax_ref[...] = jnp.maximum(elmax_ref[...], blockmax)


def _tc_prep(x, W, attn_l, attn_r):
    grid = 10
    blk = _N // grid
    return pl.pallas_call(
        _prep_body,
        grid=(grid,),
        in_specs=[
            pl.BlockSpec((blk, _IN), lambda i: (i, 0)),
            pl.BlockSpec((_H, _IN, _OUT), lambda i: (0, 0, 0)),
            pl.BlockSpec((_H, _OUT), lambda i: (0, 0)),
            pl.BlockSpec((_H, _OUT), lambda i: (0, 0)),
        ],
        out_specs=[
            pl.BlockSpec((blk, _TW), lambda i: (i, 0)),
            pl.BlockSpec((blk, 16), lambda i: (i, 0)),
            pl.BlockSpec((1, 128), lambda i: (0, 0)),
        ],
        out_shape=[
            jax.ShapeDtypeStruct((_N, _TW), jnp.float32),
            jax.ShapeDtypeStruct((_N, 16), jnp.float32),
            jax.ShapeDtypeStruct((1, 128), jnp.float32),
        ],
    )(x, W, attn_l, attn_r)


def _edge_kernel(tab, ertab, src, dst, elmax, out,
                 srcv, dstv, rows, erb, elv, acc, sem1, sem2):
    c = lax.axis_index("c")
    s = lax.axis_index("s")
    wid = c * _NS + s

    pltpu.sync_copy(elmax, elv)

    def _zrow(r, _):
        for k in range(_TW // 16):
            rows[r, pl.ds(k * 16, 16)] = jnp.zeros((16,), jnp.float32)
        return 0

    lax.fori_loop(0, _B, _zrow, 0)
    n0 = s * _RPT
    off = 0
    for sz in (128, 128, 128, 128, 113):
        pltpu.sync_copy(rows.at[pl.ds(0, sz)], acc.at[pl.ds(n0 + off, sz)])
        off += sz
    plsc.subcore_barrier()

    lanes = lax.iota(jnp.int32, 16)

    def _group(g, _):
        bvec = lanes + g * 16
        ws = []
        for h in range(_H):
            col_h = jnp.full((16,), h, jnp.int32)
            col_eh = jnp.full((16,), _IN + h, jnp.int32)
            er_d = plsc.load_gather(erb, [bvec, col_h])
            el_s = plsc.load_gather(rows, [bvec, col_eh])
            e = el_s + er_d
            e = jnp.maximum(e, 0.2 * e)
            q = elv[h] + er_d
            m = jnp.maximum(q, 0.2 * q)
            w = jnp.exp(e - m)
            plsc.store_scatter(rows, [bvec, col_eh], w)
            ws.append(w)
        for f in range(_IN):
            col_f = jnp.full((16,), f, jnp.int32)
            v = plsc.load_gather(rows, [bvec, col_f])
            plsc.store_scatter(rows, [bvec, col_f], v * ws[f // _OUT])
        return 0

    def _chunk(i, _):
        chunk = wid + i * _NW

        @pl.when(chunk < _CHUNKS)
        def _():
            eoff = chunk * _B
            pltpu.sync_copy(src.at[pl.ds(eoff, _B)], srcv)
            pltpu.sync_copy(dst.at[pl.ds(eoff, _B)], dstv)
            cp1 = pltpu.async_copy(tab.at[srcv], rows, sem1)
            cp2 = pltpu.async_copy(ertab.at[dstv], erb, sem2)
            cp1.wait()
            cp2.wait()
            lax.fori_loop(0, _B // 16, _group, 0)
            pltpu.sync_copy(rows, acc.at[dstv], add=True)

        return 0

    iters = (_CHUNKS + _NW - 1) // _NW
    lax.fori_loop(0, iters, _chunk, 0)

    plsc.subcore_barrier()
    off = 0
    for sz in (128, 128, 128, 128, 113):
        pltpu.sync_copy(acc.at[pl.ds(n0 + off, sz)],
                        out.at[c, pl.ds(n0 + off, sz)])
        off += sz


def _sc_edges(tab, ertab, src, dst, elmax16):
    mesh = plsc.VectorSubcoreMesh(core_axis_name="c", subcore_axis_name="s")
    run = functools.partial(
        pl.kernel,
        mesh=mesh,
        out_type=jax.ShapeDtypeStruct((_NC, _N, _TW), jnp.float32),
        scratch_types=[
            pltpu.VMEM((_B,), jnp.int32),
            pltpu.VMEM((_B,), jnp.int32),
            pltpu.VMEM((_B, _TW), jnp.float32),
            pltpu.VMEM((_B, 16), jnp.float32),
            pltpu.VMEM((16,), jnp.float32),
            pltpu.VMEM_SHARED((_N, _TW), jnp.float32),
            pltpu.SemaphoreType.DMA,
            pltpu.SemaphoreType.DMA,
        ],
    )(_edge_kernel)
    return run(tab, ertab, src, dst, elmax16)


def _merge_body(a_ref, b_ref, o_ref):
    a = a_ref[...]
    b = b_ref[...]
    s = a + b
    feat = s[:, :_IN]
    den = s[:, _IN:_IN + _H] + 1e-9
    den128 = jnp.concatenate(
        [jnp.broadcast_to(den[:, h:h + 1], (a.shape[0], _OUT))
         for h in range(_H)], axis=1)
    o_ref[...] = feat / den128


def _tc_merge(p0, p1):
    grid = 10
    blk = _N // grid
    return pl.pallas_call(
        _merge_body,
        grid=(grid,),
        in_specs=[
            pl.BlockSpec((blk, _TW), lambda i: (i, 0)),
            pl.BlockSpec((blk, _TW), lambda i: (i, 0)),
        ],
        out_specs=pl.BlockSpec((blk, _IN), lambda i: (i, 0)),
        out_shape=jax.ShapeDtypeStruct((_N, _IN), jnp.float32),
    )(p0, p1)


@jax.jit
def kernel(x, edge_index, W, attn_l, attn_r):
    tab, ertab, elmax = _tc_prep(x, W, attn_l, attn_r)
    elmax16 = elmax[0, :16]
    src = edge_index[0]
    dst = edge_index[1]
    partials = _sc_edges(tab, ertab, src, dst, elmax16)
    return _tc_merge(partials[0], partials[1])


# speedup vs baseline: 65.9474x; 65.9474x over previous
import functools

import jax
import jax.numpy as jnp
from jax import lax
from jax.experimental import pallas as pl
from jax.experimental.pallas import tpu as pltpu
from jax.experimental.pallas import tpu_sc as plsc

_N = 10000
_E = 320000
_IN = 128
_OUT = 32
_H = 4
_TW = _H * _OUT + 16
_B = 128
_CHUNKS = _E // _B
_NC = 2
_NS = 16
_NW = _NC * _NS
_NP = 10240
_RPT = _NP // _NS
_NEG = -3.0e38


def _prep_body(x_ref, w_ref, al_ref, ar_ref, tab_ref, er_ref, elmax_ref):
    i = pl.program_id(0)
    x = x_ref[...]
    feats = []
    els = []
    ers = []
    for h in range(_H):
        f = jnp.dot(x, w_ref[h], preferred_element_type=jnp.float32)
        feats.append(f)
        els.append(jnp.sum(f * al_ref[h][None, :], axis=1, keepdims=True))
        ers.append(jnp.sum(f * ar_ref[h][None, :], axis=1, keepdims=True))
    rows = x.shape[0]
    pad12 = jnp.zeros((rows, 12), jnp.float32)
    tab_ref[...] = jnp.concatenate(feats + els + [pad12], axis=1)
    er_ref[...] = jnp.concatenate(ers + [pad12], axis=1)

    el4 = jnp.concatenate(els, axis=1)
    padded = jnp.concatenate(
        [el4, jnp.full((rows, 124), _NEG, jnp.float32)], axis=1)
    blockmax = jnp.max(padded, axis=0, keepdims=True)

    @pl.when(i == 0)
    def _():
        elmax_ref[...] = jnp.full((1, 128), _NEG, jnp.float32)

    elmax_ref[...] = jnp.maximum(elmax_ref[...], blockmax)


def _tc_prep(x, W, attn_l, attn_r):
    grid = 10
    blk = _N // grid
    return pl.pallas_call(
        _prep_body,
        grid=(grid,),
        in_specs=[
            pl.BlockSpec((blk, _IN), lambda i: (i, 0)),
            pl.BlockSpec((_H, _IN, _OUT), lambda i: (0, 0, 0)),
            pl.BlockSpec((_H, _OUT), lambda i: (0, 0)),
            pl.BlockSpec((_H, _OUT), lambda i: (0, 0)),
        ],
        out_specs=[
            pl.BlockSpec((blk, _TW), lambda i: (i, 0)),
            pl.BlockSpec((blk, 16), lambda i: (i, 0)),
            pl.BlockSpec((1, 128), lambda i: (0, 0)),
        ],
        out_shape=[
            jax.ShapeDtypeStruct((_N, _TW), jnp.float32),
            jax.ShapeDtypeStruct((_N, 16), jnp.float32),
            jax.ShapeDtypeStruct((1, 128), jnp.float32),
        ],
    )(x, W, attn_l, attn_r)


def _edge_kernel(tab, ertab, src, dst, elmax, out,
                 srcv, dstv, rows, erb, elv, acc, sem1, sem2):
    c = lax.axis_index("c")
    s = lax.axis_index("s")
    wid = c * _NS + s

    pltpu.sync_copy(elmax, elv)

    def _zrow(r, _):
        for k in range(_TW // 16):
            rows[r, pl.ds(k * 16, 16)] = jnp.zeros((16,), jnp.float32)
        return 0

    lax.fori_loop(0, _B, _zrow, 0)
    n0 = s * _RPT
    for j in range(_RPT // _B):
        pltpu.sync_copy(rows.at[pl.ds(0, _B)], acc.at[pl.ds(n0 + j * _B, _B)])
    plsc.subcore_barrier()

    lanes = lax.iota(jnp.int32, 16)
    elvec = elv[...]
    headmask = lanes < _H

    def _edge(b, _):
        ervec = erb[b, :]
        elrow = rows[b, pl.ds(_IN, 16)]
        e = elrow + ervec
        e = jnp.maximum(e, 0.2 * e)
        q = elvec + ervec
        m = jnp.maximum(q, 0.2 * q)
        w = jnp.where(headmask, jnp.exp(e - m), 0.0)
        rows[b, pl.ds(_IN, 16)] = w
        for k in range(_IN // 16):
            wk = w[k * 16 // _OUT]
            seg = rows[b, pl.ds(k * 16, 16)]
            rows[b, pl.ds(k * 16, 16)] = seg * wk
        return 0

    def _chunk(i, _):
        chunk = wid + i * _NW

        @pl.when(chunk < _CHUNKS)
        def _():
            eoff = chunk * _B
            pltpu.sync_copy(src.at[pl.ds(eoff, _B)], srcv)
            pltpu.sync_copy(dst.at[pl.ds(eoff, _B)], dstv)
            cp1 = pltpu.async_copy(tab.at[srcv], rows, sem1)
            cp2 = pltpu.async_copy(ertab.at[dstv], erb, sem2)
            cp1.wait()
            cp2.wait()
            lax.fori_loop(0, _B, _edge, 0)
            pltpu.sync_copy(rows, acc.at[dstv], add=True)

        return 0

    iters = (_CHUNKS + _NW - 1) // _NW
    lax.fori_loop(0, iters, _chunk, 0)

    plsc.subcore_barrier()
    for j in range(_RPT // _B):
        pltpu.sync_copy(acc.at[pl.ds(n0 + j * _B, _B)],
                        out.at[c, pl.ds(n0 + j * _B, _B)])


def _sc_edges(tab, ertab, src, dst, elmax16):
    mesh = plsc.VectorSubcoreMesh(core_axis_name="c", subcore_axis_name="s")
    run = functools.partial(
        pl.kernel,
        mesh=mesh,
        compiler_params=pltpu.CompilerParams(use_tc_tiling_on_sc=False),
        out_type=jax.ShapeDtypeStruct((_NC, _NP, _TW), jnp.float32),
        scratch_types=[
            pltpu.VMEM((_B,), jnp.int32),
            pltpu.VMEM((_B,), jnp.int32),
            pltpu.VMEM((_B, _TW), jnp.float32),
            pltpu.VMEM((_B, 16), jnp.float32),
            pltpu.VMEM((16,), jnp.float32),
            pltpu.VMEM_SHARED((_NP, _TW), jnp.float32),
            pltpu.SemaphoreType.DMA,
            pltpu.SemaphoreType.DMA,
        ],
    )(_edge_kernel)
    return run(tab, ertab, src, dst, elmax16)


def _merge_body(a_ref, b_ref, o_ref):
    a = a_ref[...]
    b = b_ref[...]
    s = a + b
    feat = s[:, :_IN]
    den = s[:, _IN:_IN + _H] + 1e-9
    den128 = jnp.concatenate(
        [jnp.broadcast_to(den[:, h:h + 1], (a.shape[0], _OUT))
         for h in range(_H)], axis=1)
    o_ref[...] = feat / den128


def _tc_merge(p0, p1):
    grid = 10
    blk = _NP // grid
    return pl.pallas_call(
        _merge_body,
        grid=(grid,),
        in_specs=[
            pl.BlockSpec((blk, _TW), lambda i: (i, 0)),
            pl.BlockSpec((blk, _TW), lambda i: (i, 0)),
        ],
        out_specs=pl.BlockSpec((blk, _IN), lambda i: (i, 0)),
        out_shape=jax.ShapeDtypeStruct((_NP, _IN), jnp.float32),
    )(p0, p1)


@jax.jit
def kernel(x, edge_index, W, attn_l, attn_r):
    tab, ertab, elmax = _tc_prep(x, W, attn_l, attn_r)
    elmax16 = elmax[0, :16]
    src = edge_index[0]
    dst = edge_index[1]
    partials = _sc_edges(tab, ertab, src, dst, elmax16)
    return _tc_merge(partials[0], partials[1])[:_N]
